# baseline (speedup 1.0000x reference)
import functools

import jax
import jax.numpy as jnp
from jax import lax
from jax.experimental import pallas as pl
from jax.experimental.pallas import tpu as pltpu
from jax.experimental.pallas import tpu_sc as plsc

B, H = 4096, 200
D = 128
N = B * H
NC, NS, L = 2, 16, 16
NW = NC * NS
RPW = N // NW
CH = 128
NCHUNK = RPW // CH

HALF_PI = 1.5707963267948966
PI = 3.141592653589793
A0 = 0.9998383860193922
A1 = -0.326983305517636
A2 = 0.15936586312036266
A3 = -0.047260694565070184
SIGN_MASK = -2147483648


@functools.partial(
    pl.kernel,
    out_type=jax.ShapeDtypeStruct((N, 2 * D), jnp.float32),
    mesh=plsc.VectorSubcoreMesh(core_axis_name="c", subcore_axis_name="s"),
    scratch_types=[
        pltpu.VMEM((CH,), jnp.int32),
        pltpu.VMEM((CH, D), jnp.float32),
        pltpu.VMEM((CH, D), jnp.float32),
        pltpu.VMEM((CH, 2 * D), jnp.float32),
        pltpu.SemaphoreType.DMA,
        pltpu.SemaphoreType.DMA,
    ],
)
def _qemb(idx_hbm, real_hbm, imag_hbm, out_hbm, idx_v, re_v, im_v, ob,
          sem_r, sem_i):
    wid = lax.axis_index("s") * NC + lax.axis_index("c")
    wbase = wid * RPW

    def chunk_body(ci, carry):
        base = wbase + ci * CH
        pltpu.sync_copy(idx_hbm.at[pl.ds(base, CH)], idx_v)
        cp_r = pltpu.async_copy(real_hbm.at[idx_v], re_v, sem_r)
        cp_i = pltpu.async_copy(imag_hbm.at[idx_v], im_v, sem_i)
        cp_r.wait()
        cp_i.wait()

        def row_body(row, c2):
            for l in range(D // L):
                sl = pl.ds(l * L, L)
                r = re_v[row, sl]
                i = im_v[row, sl]
                x = r * r + i * i
                xi = lax.bitcast_convert_type(x, jnp.int32)
                y = lax.bitcast_convert_type(
                    jnp.int32(0x5F3759DF) - (xi >> 1), jnp.float32)
                y = y * (1.5 - (0.5 * x) * (y * y))
                ax = jnp.abs(r)
                ay = jnp.abs(i)
                mx = jnp.maximum(ax, ay)
                mn = jnp.minimum(ax, ay)
                nz = mx > 0.0
                mag = jnp.where(nz, x * y, 0.0)
                den = jnp.where(nz, mx, 1.0)
                t = mn / den
                u = t * t
                p = A3
                p = p * u + A2
                p = p * u + A1
                p = p * u + A0
                ph = p * t
                ph = jnp.where(ay > ax, HALF_PI - ph, ph)
                ph = jnp.where(r < 0.0, PI - ph, ph)
                ph = lax.bitcast_convert_type(
                    lax.bitcast_convert_type(ph, jnp.int32)
                    ^ (lax.bitcast_convert_type(i, jnp.int32) & SIGN_MASK),
                    jnp.float32)
                ob[row, sl] = mag
                ob[row, pl.ds(D + l * L, L)] = ph
            return c2

        lax.fori_loop(0, CH, row_body, 0, unroll=False)
        pltpu.sync_copy(ob, out_hbm.at[pl.ds(base, CH)])
        return carry

    lax.fori_loop(0, NCHUNK, chunk_body, 0, unroll=False)


def kernel(inputs, real_table, imag_table):
    idx = inputs.reshape(N).astype(jnp.int32)
    out = _qemb(idx, real_table, imag_table)
    return out.reshape(B, H, 2 * D)


# speedup vs baseline: 2.7608x; 2.7608x over previous
import functools

import jax
import jax.numpy as jnp
from jax import lax
from jax.experimental import pallas as pl
from jax.experimental.pallas import tpu as pltpu
from jax.experimental.pallas import tpu_sc as plsc

B, H = 4096, 200
D = 128
N = B * H
NC, NS, L = 2, 16, 16
NW = NC * NS
RPW = N // NW
CH = 128
NCHUNK = RPW // CH

HALF_PI = 1.5707963267948966
PI = 3.141592653589793
A0 = 0.9998383860193922
A1 = -0.326983305517636
A2 = 0.15936586312036266
A3 = -0.047260694565070184
SIGN_MASK = -2147483648


@functools.partial(
    pl.kernel,
    out_type=jax.ShapeDtypeStruct((N, 2 * D), jnp.float32),
    mesh=plsc.VectorSubcoreMesh(core_axis_name="c", subcore_axis_name="s"),
    scratch_types=[
        pltpu.VMEM((CH,), jnp.int32),
        pltpu.VMEM((CH, D), jnp.float32),
        pltpu.VMEM((CH, D), jnp.float32),
        pltpu.SemaphoreType.DMA,
        pltpu.SemaphoreType.DMA,
    ],
)
def _qemb(idx_hbm, real_hbm, imag_hbm, out_hbm, idx_v, re_v, im_v,
          sem_r, sem_i):
    wid = lax.axis_index("s") * NC + lax.axis_index("c")
    wbase = wid * RPW

    def chunk_body(ci, carry):
        base = wbase + ci * CH
        pltpu.sync_copy(idx_hbm.at[pl.ds(base, CH)], idx_v)
        cp_r = pltpu.async_copy(real_hbm.at[idx_v], re_v, sem_r)
        cp_i = pltpu.async_copy(imag_hbm.at[idx_v], im_v, sem_i)
        cp_r.wait()
        cp_i.wait()

        def row_body(row, c2):
            for l in range(D // L):
                sl = pl.ds(l * L, L)
                r = re_v[row, sl]
                i = im_v[row, sl]
                x = r * r + i * i
                xi = lax.bitcast_convert_type(x, jnp.int32)
                y = lax.bitcast_convert_type(
                    jnp.int32(0x5F3759DF) - (xi >> 1), jnp.float32)
                y = y * (1.5 - (0.5 * x) * (y * y))
                ax = jnp.abs(r)
                ay = jnp.abs(i)
                mx = jnp.maximum(ax, ay)
                mn = jnp.minimum(ax, ay)
                nz = mx > 0.0
                mag = jnp.where(nz, x * y, 0.0)
                den = jnp.where(nz, mx, 1.0)
                t = mn / den
                u = t * t
                p = A3
                p = p * u + A2
                p = p * u + A1
                p = p * u + A0
                ph = p * t
                ph = jnp.where(ay > ax, HALF_PI - ph, ph)
                ph = jnp.where(r < 0.0, PI - ph, ph)
                ph = lax.bitcast_convert_type(
                    lax.bitcast_convert_type(ph, jnp.int32)
                    ^ (lax.bitcast_convert_type(i, jnp.int32) & SIGN_MASK),
                    jnp.float32)
                re_v[row, sl] = mag
                im_v[row, sl] = ph
            return c2

        lax.fori_loop(0, CH, row_body, 0, unroll=False)
        pltpu.sync_copy(re_v, out_hbm.at[pl.ds(base, CH), pl.ds(0, D)])
        pltpu.sync_copy(im_v, out_hbm.at[pl.ds(base, CH), pl.ds(D, D)])
        return carry

    lax.fori_loop(0, NCHUNK, chunk_body, 0, unroll=False)


def kernel(inputs, real_table, imag_table):
    idx = inputs.reshape(N).astype(jnp.int32)
    out = _qemb(idx, real_table, imag_table)
    return out.reshape(B, H, 2 * D)


# speedup vs baseline: 4.1745x; 1.5120x over previous
import functools

import jax
import jax.numpy as jnp
from jax import lax
from jax.experimental import pallas as pl
from jax.experimental.pallas import tpu as pltpu
from jax.experimental.pallas import tpu_sc as plsc

B, H = 4096, 200
D = 128
N = B * H
NC, NS, L = 2, 16, 16
NW = NC * NS
RPW = N // NW
CH = 128
NCHUNK = RPW // CH
NB = 3

HALF_PI = 1.5707963267948966
PI = 3.141592653589793
A0 = 0.9998383860193922
A1 = -0.326983305517636
A2 = 0.15936586312036266
A3 = -0.047260694565070184
SIGN_MASK = -2147483648


@functools.partial(
    pl.kernel,
    out_type=jax.ShapeDtypeStruct((N, 2 * D), jnp.float32),
    mesh=plsc.VectorSubcoreMesh(core_axis_name="c", subcore_axis_name="s"),
    scratch_types=[
        pltpu.VMEM((RPW,), jnp.int32),
        pltpu.VMEM((NB, CH, D), jnp.float32),
        pltpu.VMEM((NB, CH, D), jnp.float32),
        pltpu.SemaphoreType.DMA,
        pltpu.SemaphoreType.DMA,
        pltpu.SemaphoreType.DMA,
        pltpu.SemaphoreType.DMA,
        pltpu.SemaphoreType.DMA,
        pltpu.SemaphoreType.DMA,
    ],
)
def _qemb(idx_hbm, real_hbm, imag_hbm, out_hbm, idx_all, re_v, im_v,
          sg0, sg1, sg2, sw0, sw1, sw2):
    sem_g = (sg0, sg1, sg2)
    sem_w = (sw0, sw1, sw2)
    wid = lax.axis_index("s") * NC + lax.axis_index("c")
    wbase = wid * RPW

    pltpu.sync_copy(idx_hbm.at[wid], idx_all)

    def fire_gather(ci, b):
        ix = idx_all.at[pl.ds(ci * CH, CH)]
        pltpu.async_copy(real_hbm.at[ix], re_v.at[b], sem_g[b])
        pltpu.async_copy(imag_hbm.at[ix], im_v.at[b], sem_g[b])

    def wait_gather(ci, b):
        ix = idx_all.at[pl.ds(ci * CH, CH)]
        pltpu.make_async_copy(real_hbm.at[ix], re_v.at[b], sem_g[b]).wait()
        pltpu.make_async_copy(imag_hbm.at[ix], im_v.at[b], sem_g[b]).wait()

    def fire_write(ci, b):
        base = wbase + ci * CH
        pltpu.async_copy(
            re_v.at[b], out_hbm.at[pl.ds(base, CH), pl.ds(0, D)], sem_w[b])
        pltpu.async_copy(
            im_v.at[b], out_hbm.at[pl.ds(base, CH), pl.ds(D, D)], sem_w[b])

    def wait_write(ci, b):
        base = wbase + ci * CH
        pltpu.make_async_copy(
            re_v.at[b], out_hbm.at[pl.ds(base, CH), pl.ds(0, D)],
            sem_w[b]).wait()
        pltpu.make_async_copy(
            im_v.at[b], out_hbm.at[pl.ds(base, CH), pl.ds(D, D)],
            sem_w[b]).wait()

    def compute(b):
        def row_body(row, c2):
            for l in range(D // L):
                sl = pl.ds(l * L, L)
                r = re_v[b, row, sl]
                i = im_v[b, row, sl]
                x = r * r + i * i
                xi = lax.bitcast_convert_type(x, jnp.int32)
                y = lax.bitcast_convert_type(
                    jnp.int32(0x5F3759DF) - (xi >> 1), jnp.float32)
                y = y * (1.5 - (0.5 * x) * (y * y))
                ax = jnp.abs(r)
                ay = jnp.abs(i)
                mx = jnp.maximum(ax, ay)
                mn = jnp.minimum(ax, ay)
                nz = mx > 0.0
                mag = jnp.where(nz, x * y, 0.0)
                den = jnp.where(nz, mx, 1.0)
                t = mn / den
                u = t * t
                p = A3
                p = p * u + A2
                p = p * u + A1
                p = p * u + A0
                ph = p * t
                ph = jnp.where(ay > ax, HALF_PI - ph, ph)
                ph = jnp.where(r < 0.0, PI - ph, ph)
                ph = lax.bitcast_convert_type(
                    lax.bitcast_convert_type(ph, jnp.int32)
                    ^ (lax.bitcast_convert_type(i, jnp.int32) & SIGN_MASK),
                    jnp.float32)
                re_v[b, row, sl] = mag
                im_v[b, row, sl] = ph
            return c2

        lax.fori_loop(0, CH, row_body, 0, unroll=False)

    def substep(ci, b, wait_prev_write, fire_next):
        wait_gather(ci, b)
        compute(b)
        fire_write(ci, b)
        if wait_prev_write:
            wait_write(ci - 1, (b + 2) % NB)
        if fire_next:
            fire_gather(ci + 2, (b + 2) % NB)

    fire_gather(0, 0)
    fire_gather(1, 1)

    substep(0, 0, False, True)
    substep(1, 1, True, True)
    substep(2, 2, True, True)

    def turn(k, carry):
        ci = k * NB
        for b in range(NB):
            substep(ci + b, b, True, True)
        return carry

    lax.fori_loop(1, (NCHUNK - 2) // NB, turn, 0, unroll=False)

    substep(NCHUNK - 2, 0, True, False)
    substep(NCHUNK - 1, 1, True, False)
    wait_write(NCHUNK - 1, 1)


def kernel(inputs, real_table, imag_table):
    idx = inputs.reshape(NW, RPW).astype(jnp.int32)
    out = _qemb(idx, real_table, imag_table)
    return out.reshape(B, H, 2 * D)


# speedup vs baseline: 4.6324x; 1.1097x over previous
import functools

import jax
import jax.numpy as jnp
from jax import lax
from jax.experimental import pallas as pl
from jax.experimental.pallas import tpu as pltpu
from jax.experimental.pallas import tpu_sc as plsc

B, H = 4096, 200
D = 128
N = B * H
NC, NS, L = 2, 16, 16
NW = NC * NS
RPW = N // NW
CH = 128
NCHUNK = RPW // CH
NB = 3

HALF_PI = 1.5707963267948966
PI = 3.141592653589793
A0 = 0.9998383860193922
A1 = -0.326983305517636
A2 = 0.15936586312036266
A3 = -0.047260694565070184
SIGN_MASK = -2147483648


@functools.partial(
    pl.kernel,
    out_type=jax.ShapeDtypeStruct((N, 2 * D), jnp.float32),
    mesh=plsc.VectorSubcoreMesh(core_axis_name="c", subcore_axis_name="s"),
    scratch_types=[
        pltpu.VMEM((RPW,), jnp.int32),
        pltpu.VMEM((NB, CH, D), jnp.float32),
        pltpu.VMEM((NB, CH, D), jnp.float32),
        pltpu.SemaphoreType.DMA,
        pltpu.SemaphoreType.DMA,
        pltpu.SemaphoreType.DMA,
        pltpu.SemaphoreType.DMA,
        pltpu.SemaphoreType.DMA,
        pltpu.SemaphoreType.DMA,
    ],
)
def _qemb(idx_hbm, real_hbm, imag_hbm, out_hbm, idx_all, re_v, im_v,
          sg0, sg1, sg2, sw0, sw1, sw2):
    sem_g = (sg0, sg1, sg2)
    sem_w = (sw0, sw1, sw2)
    wid = lax.axis_index("s") * NC + lax.axis_index("c")
    wbase = wid * RPW

    pltpu.sync_copy(idx_hbm.at[wid], idx_all)

    def fire_gather(ci, b):
        ix = idx_all.at[pl.ds(ci * CH, CH)]
        pltpu.async_copy(real_hbm.at[ix], re_v.at[b], sem_g[b])
        pltpu.async_copy(imag_hbm.at[ix], im_v.at[b], sem_g[b])

    def wait_gather(ci, b):
        ix = idx_all.at[pl.ds(ci * CH, CH)]
        pltpu.make_async_copy(real_hbm.at[ix], re_v.at[b], sem_g[b]).wait()
        pltpu.make_async_copy(imag_hbm.at[ix], im_v.at[b], sem_g[b]).wait()

    def fire_write(ci, b):
        base = wbase + ci * CH
        pltpu.async_copy(
            re_v.at[b], out_hbm.at[pl.ds(base, CH), pl.ds(0, D)], sem_w[b])
        pltpu.async_copy(
            im_v.at[b], out_hbm.at[pl.ds(base, CH), pl.ds(D, D)], sem_w[b])

    def wait_write(ci, b):
        base = wbase + ci * CH
        pltpu.make_async_copy(
            re_v.at[b], out_hbm.at[pl.ds(base, CH), pl.ds(0, D)],
            sem_w[b]).wait()
        pltpu.make_async_copy(
            im_v.at[b], out_hbm.at[pl.ds(base, CH), pl.ds(D, D)],
            sem_w[b]).wait()

    def compute(b):
        def row_body(row, c2):
            for l in range(D // L):
                sl = pl.ds(l * L, L)
                r = re_v[b, row, sl]
                i = im_v[b, row, sl]
                ax = jnp.abs(r)
                ay = jnp.abs(i)
                mx = jnp.maximum(ax, ay)
                mn = jnp.minimum(ax, ay)
                den = jnp.maximum(mx, 1e-38)
                t = mn / den
                u = t * t
                s = 1.0 + u
                si = lax.bitcast_convert_type(s, jnp.int32)
                y = lax.bitcast_convert_type(
                    jnp.int32(0x5F3759DF) - (si >> 1), jnp.float32)
                y = y * (1.5 - (0.5 * s) * (y * y))
                mag = (mx * s) * y
                p = A3
                p = p * u + A2
                p = p * u + A1
                p = p * u + A0
                ph = p * t
                ph = jnp.where(ay > ax, HALF_PI - ph, ph)
                ph = jnp.where(r < 0.0, PI - ph, ph)
                ph = lax.bitcast_convert_type(
                    lax.bitcast_convert_type(ph, jnp.int32)
                    ^ (lax.bitcast_convert_type(i, jnp.int32) & SIGN_MASK),
                    jnp.float32)
                re_v[b, row, sl] = mag
                im_v[b, row, sl] = ph
            return c2

        lax.fori_loop(0, CH, row_body, 0, unroll=False)

    def substep(ci, b, wait_prev_write, fire_next):
        wait_gather(ci, b)
        compute(b)
        fire_write(ci, b)
        if wait_prev_write:
            wait_write(ci - 1, (b + 2) % NB)
        if fire_next:
            fire_gather(ci + 2, (b + 2) % NB)

    fire_gather(0, 0)
    fire_gather(1, 1)

    substep(0, 0, False, True)
    substep(1, 1, True, True)
    substep(2, 2, True, True)

    def turn(k, carry):
        ci = k * NB
        for b in range(NB):
            substep(ci + b, b, True, True)
        return carry

    lax.fori_loop(1, (NCHUNK - 2) // NB, turn, 0, unroll=False)

    substep(NCHUNK - 2, 0, True, False)
    substep(NCHUNK - 1, 1, True, False)
    wait_write(NCHUNK - 1, 1)


def kernel(inputs, real_table, imag_table):
    idx = inputs.reshape(NW, RPW).astype(jnp.int32)
    out = _qemb(idx, real_table, imag_table)
    return out.reshape(B, H, 2 * D)


# speedup vs baseline: 7.6689x; 1.6555x over previous
import functools

import jax
import jax.numpy as jnp
from jax import lax
from jax.experimental import pallas as pl
from jax.experimental.pallas import tpu as pltpu
from jax.experimental.pallas import tpu_sc as plsc

B, H = 4096, 200
D = 128
N = B * H
V = 100000
NC, NS, L = 2, 16, 16
NW = NC * NS
RPW = N // NW
CH = 128
NCHUNK = RPW // CH
NB = 3
VPW = 3128
TCH = 136
TNCH = VPW // TCH

HALF_PI = 1.5707963267948966
PI = 3.141592653589793
A0 = 0.9998383860193922
A1 = -0.326983305517636
A2 = 0.15936586312036266
A3 = -0.047260694565070184
SIGN_MASK = -2147483648


def _magphase(r, i):
    ax = jnp.abs(r)
    ay = jnp.abs(i)
    mx = jnp.maximum(ax, ay)
    mn = jnp.minimum(ax, ay)
    den = jnp.maximum(mx, 1e-38)
    t = mn / den
    u = t * t
    s = 1.0 + u
    si = lax.bitcast_convert_type(s, jnp.int32)
    y = lax.bitcast_convert_type(
        jnp.int32(0x5F3759DF) - (si >> 1), jnp.float32)
    y = y * (1.5 - (0.5 * s) * (y * y))
    mag = (mx * s) * y
    p = A3
    p = p * u + A2
    p = p * u + A1
    p = p * u + A0
    ph = p * t
    ph = jnp.where(ay > ax, HALF_PI - ph, ph)
    ph = jnp.where(r < 0.0, PI - ph, ph)
    ph = lax.bitcast_convert_type(
        lax.bitcast_convert_type(ph, jnp.int32)
        ^ (lax.bitcast_convert_type(i, jnp.int32) & SIGN_MASK),
        jnp.float32)
    return mag, ph


@functools.partial(
    pl.kernel,
    out_type=(jax.ShapeDtypeStruct((V, D), jnp.float32),
              jax.ShapeDtypeStruct((V, D), jnp.float32)),
    mesh=plsc.VectorSubcoreMesh(core_axis_name="c", subcore_axis_name="s"),
    scratch_types=[
        pltpu.VMEM((2, TCH, D), jnp.float32),
        pltpu.VMEM((2, TCH, D), jnp.float32),
        pltpu.SemaphoreType.DMA,
        pltpu.SemaphoreType.DMA,
        pltpu.SemaphoreType.DMA,
        pltpu.SemaphoreType.DMA,
    ],
)
def _transform(real_hbm, imag_hbm, mag_hbm, ph_hbm, re_v, im_v,
               sg0, sg1, sw0, sw1):
    sem_g = (sg0, sg1)
    sem_w = (sw0, sw1)
    wid = lax.axis_index("s") * NC + lax.axis_index("c")
    wbase = jnp.minimum(wid * VPW, V - VPW)

    def fire_read(ci, b):
        base = wbase + ci * TCH
        pltpu.async_copy(real_hbm.at[pl.ds(base, TCH)], re_v.at[b], sem_g[b])
        pltpu.async_copy(imag_hbm.at[pl.ds(base, TCH)], im_v.at[b], sem_g[b])

    def wait_read(ci, b):
        base = wbase + ci * TCH
        pltpu.make_async_copy(
            real_hbm.at[pl.ds(base, TCH)], re_v.at[b], sem_g[b]).wait()
        pltpu.make_async_copy(
            imag_hbm.at[pl.ds(base, TCH)], im_v.at[b], sem_g[b]).wait()

    def fire_write(ci, b):
        base = wbase + ci * TCH
        pltpu.async_copy(re_v.at[b], mag_hbm.at[pl.ds(base, TCH)], sem_w[b])
        pltpu.async_copy(im_v.at[b], ph_hbm.at[pl.ds(base, TCH)], sem_w[b])

    def wait_write(ci, b):
        base = wbase + ci * TCH
        pltpu.make_async_copy(
            re_v.at[b], mag_hbm.at[pl.ds(base, TCH)], sem_w[b]).wait()
        pltpu.make_async_copy(
            im_v.at[b], ph_hbm.at[pl.ds(base, TCH)], sem_w[b]).wait()

    def compute(b):
        def row_body(row, c2):
            for l in range(D // L):
                sl = pl.ds(l * L, L)
                mag, ph = _magphase(re_v[b, row, sl], im_v[b, row, sl])
                re_v[b, row, sl] = mag
                im_v[b, row, sl] = ph
            return c2

        lax.fori_loop(0, TCH, row_body, 0, unroll=False)

    def substep(ci, b, wait_prev_write, fire_next):
        wait_read(ci, b)
        if fire_next:
            if wait_prev_write:
                wait_write(ci - 1, (b + 1) % 2)
            fire_read(ci + 1, (b + 1) % 2)
        compute(b)
        fire_write(ci, b)

    fire_read(0, 0)
    substep(0, 0, False, True)

    def turn(k, carry):
        ci = 2 * k - 1
        substep(ci, 1, True, True)
        substep(ci + 1, 0, True, True)
        return carry

    lax.fori_loop(1, TNCH // 2, turn, 0, unroll=False)

    substep(TNCH - 2, 1, True, True)
    substep(TNCH - 1, 0, True, False)
    wait_write(TNCH - 2, 1)
    wait_write(TNCH - 1, 0)


@functools.partial(
    pl.kernel,
    out_type=jax.ShapeDtypeStruct((N, 2 * D), jnp.float32),
    mesh=plsc.VectorSubcoreMesh(core_axis_name="c", subcore_axis_name="s"),
    scratch_types=[
        pltpu.VMEM((RPW,), jnp.int32),
        pltpu.VMEM((NB, CH, D), jnp.float32),
        pltpu.VMEM((NB, CH, D), jnp.float32),
        pltpu.SemaphoreType.DMA,
        pltpu.SemaphoreType.DMA,
        pltpu.SemaphoreType.DMA,
        pltpu.SemaphoreType.DMA,
        pltpu.SemaphoreType.DMA,
        pltpu.SemaphoreType.DMA,
    ],
)
def _qgather(idx_hbm, mag_hbm, ph_hbm, out_hbm, idx_all, mg_v, ph_v,
             sg0, sg1, sg2, sw0, sw1, sw2):
    sem_g = (sg0, sg1, sg2)
    sem_w = (sw0, sw1, sw2)
    wid = lax.axis_index("s") * NC + lax.axis_index("c")
    wbase = wid * RPW

    pltpu.sync_copy(idx_hbm.at[wid], idx_all)

    def fire_gather(ci, b):
        ix = idx_all.at[pl.ds(ci * CH, CH)]
        pltpu.async_copy(mag_hbm.at[ix], mg_v.at[b], sem_g[b])
        pltpu.async_copy(ph_hbm.at[ix], ph_v.at[b], sem_g[b])

    def wait_gather(ci, b):
        ix = idx_all.at[pl.ds(ci * CH, CH)]
        pltpu.make_async_copy(mag_hbm.at[ix], mg_v.at[b], sem_g[b]).wait()
        pltpu.make_async_copy(ph_hbm.at[ix], ph_v.at[b], sem_g[b]).wait()

    def fire_write(ci, b):
        base = wbase + ci * CH
        pltpu.async_copy(
            mg_v.at[b], out_hbm.at[pl.ds(base, CH), pl.ds(0, D)], sem_w[b])
        pltpu.async_copy(
            ph_v.at[b], out_hbm.at[pl.ds(base, CH), pl.ds(D, D)], sem_w[b])

    def wait_write(ci, b):
        base = wbase + ci * CH
        pltpu.make_async_copy(
            mg_v.at[b], out_hbm.at[pl.ds(base, CH), pl.ds(0, D)],
            sem_w[b]).wait()
        pltpu.make_async_copy(
            ph_v.at[b], out_hbm.at[pl.ds(base, CH), pl.ds(D, D)],
            sem_w[b]).wait()

    def substep(ci, b, wait_prev_write, fire_next):
        wait_gather(ci, b)
        fire_write(ci, b)
        if wait_prev_write:
            wait_write(ci - 1, (b + 2) % NB)
        if fire_next:
            fire_gather(ci + 2, (b + 2) % NB)

    fire_gather(0, 0)
    fire_gather(1, 1)

    substep(0, 0, False, True)
    substep(1, 1, True, True)
    substep(2, 2, True, True)

    def turn(k, carry):
        ci = k * NB
        for b in range(NB):
            substep(ci + b, b, True, True)
        return carry

    lax.fori_loop(1, (NCHUNK - 2) // NB, turn, 0, unroll=False)

    substep(NCHUNK - 2, 0, True, False)
    substep(NCHUNK - 1, 1, True, False)
    wait_write(NCHUNK - 1, 1)


def kernel(inputs, real_table, imag_table):
    idx = inputs.reshape(NW, RPW).astype(jnp.int32)
    mag_t, ph_t = _transform(real_table, imag_table)
    out = _qgather(idx, mag_t, ph_t)
    return out.reshape(B, H, 2 * D)
